# baseline (speedup 1.0000x reference)
import functools

import jax
import jax.numpy as jnp
from jax import lax
from jax.experimental import pallas as pl
from jax.experimental.pallas import tpu as pltpu
from jax.experimental.pallas import tpu_sc as plsc

_NC = 2
_NS = 16
_NW = _NC * _NS
_L = 16

_N = 50000
_E = 800000
_RT = 3136
_RC = 784
_NPAD = _NS * _RT
_RLAST = _N - (_NS - 1) * _RT - 3 * _RC

_NBLK = _E // 128
_CB = 8
_NQ = -(-_NBLK // _CB)
_CBT = _NBLK - (_NQ - 1) * _CB
_NF = _NQ - 1
_WTAIL = _NF % _NW


def _sc_body(a_hbm, b_hbm, x_hbm, e_hbm, nv_hbm, st_hbm, wtab_hbm,
             av, bv, xv, wrow, nvf, idx0, idx1, r30, r31, obf, obft,
             sem0, sem1):
  cid = lax.axis_index("c")
  sid = lax.axis_index("s")
  wid = sid * _NC + cid
  lane = lax.iota(jnp.int32, _L)

  def build_nv(row_base, nrows):
    pltpu.sync_copy(a_hbm.at[pl.ds(row_base * 9, nrows * 9)],
                    av.at[pl.ds(0, nrows * 9)])
    pltpu.sync_copy(b_hbm.at[pl.ds(row_base * 3, nrows * 3)],
                    bv.at[pl.ds(0, nrows * 3)])
    pltpu.sync_copy(x_hbm.at[pl.ds(row_base * 3, nrows * 3)],
                    xv.at[pl.ds(0, nrows * 3)])

    @plsc.parallel_loop(0, nrows // _L)
    def group(g):
      nid = g * _L + lane
      xs = [plsc.load_gather(xv, [nid * 3 + j]) for j in range(3)]
      for i in range(3):
        bi = plsc.load_gather(bv, [nid * 3 + i])
        plsc.store_scatter(wrow, [nid, jnp.full((_L,), 9 + i, jnp.int32)], bi)
        acc = bi
        for j in range(3):
          aij = plsc.load_gather(av, [nid * 9 + 3 * i + j])
          plsc.store_scatter(
              wrow, [nid, jnp.full((_L,), 3 * i + j, jnp.int32)], aij)
          acc = acc + aij * xs[j]
        plsc.store_scatter(nvf, [nid * 3 + i], acc)
    pltpu.sync_copy(wrow.at[pl.ds(0, nrows)],
                    wtab_hbm.at[pl.ds(row_base, nrows)])
    pltpu.sync_copy(nvf.at[pl.ds(0, nrows * 3)],
                    nv_hbm.at[pl.ds(row_base * 3, nrows * 3)])

  with jax.named_scope("p1_build"):
    for r in range(3):
      build_nv(sid * _RT + r * _RC, _RC)

    @pl.when(sid < _NS - 1)
    def _():
      build_nv(sid * _RT + 3 * _RC, _RC)

    @pl.when(sid == _NS - 1)
    def _():
      build_nv(sid * _RT + 3 * _RC, _RLAST)

  with jax.named_scope("p1_barrier"):
    plsc.subcore_barrier()

  def stage(q, idx, sem, rbuf):
    pltpu.sync_copy(e_hbm.at[pl.ds(q * (_CB * 256), _CB * 256)], idx)
    pltpu.async_copy(wtab_hbm.at[idx], rbuf, sem)

  def compute(rbuf, obuf, nblk):
    @plsc.parallel_loop(0, 8, unroll=2)
    def col(ep0):
      for blk in range(nblk):
        rv0 = lane + (blk * 256 + ep0 * 16)
        rv1 = rv0 + 128
        for c in range(12):
          i, j = (c // 3, c % 3) if c < 9 else (c - 9, 3)
          cv = jnp.full((_L,), c, jnp.int32)
          d = (plsc.load_gather(rbuf, [rv0, cv])
               - plsc.load_gather(rbuf, [rv1, cv]))
          obuf[i, pl.ds(blk * 512 + j * 128 + ep0 * 16, _L)] = d * d

  nf = (_NF - wid + _NW - 1) // _NW

  @pl.when(nf > 0)
  def _():
    stage(wid, idx0, sem0, r30)

  def chunk(k, carry):
    q = wid + _NW * k

    def run(idx, sem, rbuf, idxn, semn, rbufn):
      with jax.named_scope("p2_wait"):
        pltpu.make_async_copy(wtab_hbm.at[idx], rbuf, sem).wait()

      with jax.named_scope("p2_stage"):
        @pl.when(k + 1 < nf)
        def _():
          stage(q + _NW, idxn, semn, rbufn)

      with jax.named_scope("p2_compute"):
        compute(rbuf, obf, _CB)
      with jax.named_scope("p2_out"):
        pltpu.sync_copy(obf, st_hbm.at[:, pl.ds(q * (_CB * 512), _CB * 512)])

    @pl.when(k % 2 == 0)
    def _():
      run(idx0, sem0, r30, idx1, sem1, r31)

    @pl.when(k % 2 == 1)
    def _():
      run(idx1, sem1, r31, idx0, sem0, r30)

    return carry

  lax.fori_loop(0, nf, chunk, 0)

  @pl.when(wid == _WTAIL)
  def _():
    pltpu.sync_copy(e_hbm.at[pl.ds(_NF * (_CB * 256), _CBT * 256)],
                    idx0.at[pl.ds(0, _CBT * 256)])
    pltpu.async_copy(wtab_hbm.at[idx0.at[pl.ds(0, _CBT * 256)]],
                     r30.at[pl.ds(0, _CBT * 256)], sem0).wait()
    compute(r30, obft, _CBT)
    pltpu.sync_copy(obft,
                    st_hbm.at[:, pl.ds(_NF * (_CB * 512), _CBT * 512)])


_sc_kernel = functools.partial(
    pl.kernel,
    out_type=(
        jax.ShapeDtypeStruct((_N * 3,), jnp.float32),
        jax.ShapeDtypeStruct((3, _NBLK * 512), jnp.float32),
        jax.ShapeDtypeStruct((_NPAD, 16), jnp.float32),
    ),
    mesh=plsc.VectorSubcoreMesh(
        core_axis_name="c", subcore_axis_name="s",
        num_cores=_NC, num_subcores=_NS),
    compiler_params=pltpu.CompilerParams(
        needs_layout_passes=False, use_tc_tiling_on_sc=False),
    scratch_types=[
        pltpu.VMEM((_RC * 9,), jnp.float32),
        pltpu.VMEM((_RC * 3,), jnp.float32),
        pltpu.VMEM((_RC * 3,), jnp.float32),
        pltpu.VMEM((_RC, 16), jnp.float32),
        pltpu.VMEM((_RC * 3,), jnp.float32),
        pltpu.VMEM((_CB * 256,), jnp.int32),
        pltpu.VMEM((_CB * 256,), jnp.int32),
        pltpu.VMEM((_CB * 256, 16), jnp.float32),
        pltpu.VMEM((_CB * 256, 16), jnp.float32),
        pltpu.VMEM((3, _CB * 512), jnp.float32),
        pltpu.VMEM((3, _CBT * 512), jnp.float32),
        pltpu.SemaphoreType.DMA,
        pltpu.SemaphoreType.DMA,
    ],
)(_sc_body)


def kernel(x, edges, A, b):
  B, N, _ = x.shape
  E = edges.shape[0]
  ev = edges.astype(jnp.int32).reshape(_NBLK, 128, 2)
  ev = ev.transpose(0, 2, 1).reshape(_NBLK * 256)
  nv, st, _unused_tab = _sc_kernel(
      A.reshape(N * 9), b.reshape(N * 3), x.reshape(N * 3), ev)
  stiffness = (st.reshape(3, _NBLK, 4, 128)
               .transpose(1, 3, 0, 2).reshape(B, E, 3, 4))
  return (nv.reshape(B, N, 3), stiffness)


# speedup vs baseline: 1.7289x; 1.7289x over previous
import functools

import jax
import jax.numpy as jnp
from jax import lax
from jax.experimental import pallas as pl
from jax.experimental.pallas import tpu as pltpu
from jax.experimental.pallas import tpu_sc as plsc

_NC = 2
_NS = 16
_NW = _NC * _NS
_L = 16

_N = 50000
_E = 800000
_RT = 3136
_RC = 784
_NPAD = _NS * _RT
_RLAST = _N - (_NS - 1) * _RT - 3 * _RC

_NBLK = _E // 128
_CB = 8
_NQ = -(-_NBLK // _CB)
_CBT = _NBLK - (_NQ - 1) * _CB
_NF = _NQ - 1
_WTAIL = _NF % _NW


def _sc_body(a_hbm, b_hbm, x_hbm, e_hbm, nv_hbm, st_hbm, wtab_hbm,
             av, bv, xv, wrow, nvf, idx0, idx1, r30, r31, obf, obft,
             sem0, sem1):
  cid = lax.axis_index("c")
  sid = lax.axis_index("s")
  wid = sid * _NC + cid
  lane = lax.iota(jnp.int32, _L)

  def build_nv(row_base, nrows):
    for c in range(9):
      pltpu.sync_copy(a_hbm.at[pl.ds(c * _N + row_base, nrows)],
                      av.at[pl.ds(c * _RC, nrows)])
    for c in range(3):
      pltpu.sync_copy(b_hbm.at[pl.ds(c * _N + row_base, nrows)],
                      bv.at[pl.ds(c * _RC, nrows)])
      pltpu.sync_copy(x_hbm.at[pl.ds(c * _N + row_base, nrows)],
                      xv.at[pl.ds(c * _RC, nrows)])

    @plsc.parallel_loop(0, nrows // _L)
    def group(g):
      nid = g * _L + lane
      xs = [xv[pl.ds(j * _RC + g * _L, _L)] for j in range(3)]
      for i in range(3):
        bi = bv[pl.ds(i * _RC + g * _L, _L)]
        plsc.store_scatter(wrow, [nid, jnp.full((_L,), 9 + i, jnp.int32)], bi)
        acc = bi
        for j in range(3):
          aij = av[pl.ds((3 * i + j) * _RC + g * _L, _L)]
          plsc.store_scatter(
              wrow, [nid, jnp.full((_L,), 3 * i + j, jnp.int32)], aij)
          acc = acc + aij * xs[j]
        nvf[pl.ds(i * _RC + g * _L, _L)] = acc
    pltpu.sync_copy(wrow.at[pl.ds(0, nrows)],
                    wtab_hbm.at[pl.ds(row_base, nrows)])
    for i in range(3):
      pltpu.sync_copy(nvf.at[pl.ds(i * _RC, nrows)],
                      nv_hbm.at[pl.ds(i * _N + row_base, nrows)])

  with jax.named_scope("p1_build"):
    for r in range(3):
      build_nv(sid * _RT + r * _RC, _RC)

    @pl.when(sid < _NS - 1)
    def _():
      build_nv(sid * _RT + 3 * _RC, _RC)

    @pl.when(sid == _NS - 1)
    def _():
      build_nv(sid * _RT + 3 * _RC, _RLAST)

  with jax.named_scope("p1_barrier"):
    plsc.subcore_barrier()

  def stage(q, idx, sem, rbuf):
    pltpu.sync_copy(e_hbm.at[pl.ds(q * (_CB * 256), _CB * 256)], idx)
    pltpu.async_copy(wtab_hbm.at[idx], rbuf, sem)

  def compute(rbuf, obuf, nblk):
    @plsc.parallel_loop(0, 8)
    def col(ep0):
      for blk in range(nblk):
        rv0 = lane + (blk * 256 + ep0 * 16)
        rv1 = rv0 + 128
        for c in range(12):
          i, j = (c // 3, c % 3) if c < 9 else (c - 9, 3)
          cv = jnp.full((_L,), c, jnp.int32)
          d = (plsc.load_gather(rbuf, [rv0, cv])
               - plsc.load_gather(rbuf, [rv1, cv]))
          obuf[i, pl.ds(blk * 512 + j * 128 + ep0 * 16, _L)] = d * d

  nf = (_NF - wid + _NW - 1) // _NW

  @pl.when(nf > 0)
  def _():
    stage(wid, idx0, sem0, r30)

  def chunk(k, carry):
    q = wid + _NW * k

    def run(idx, sem, rbuf, idxn, semn, rbufn):
      with jax.named_scope("p2_wait"):
        pltpu.make_async_copy(wtab_hbm.at[idx], rbuf, sem).wait()

      with jax.named_scope("p2_stage"):
        @pl.when(k + 1 < nf)
        def _():
          stage(q + _NW, idxn, semn, rbufn)

      with jax.named_scope("p2_compute"):
        compute(rbuf, obf, _CB)
      with jax.named_scope("p2_out"):
        pltpu.sync_copy(obf, st_hbm.at[:, pl.ds(q * (_CB * 512), _CB * 512)])

    @pl.when(k % 2 == 0)
    def _():
      run(idx0, sem0, r30, idx1, sem1, r31)

    @pl.when(k % 2 == 1)
    def _():
      run(idx1, sem1, r31, idx0, sem0, r30)

    return carry

  lax.fori_loop(0, nf, chunk, 0)

  @pl.when(wid == _WTAIL)
  def _():
    pltpu.sync_copy(e_hbm.at[pl.ds(_NF * (_CB * 256), _CBT * 256)],
                    idx0.at[pl.ds(0, _CBT * 256)])
    pltpu.async_copy(wtab_hbm.at[idx0.at[pl.ds(0, _CBT * 256)]],
                     r30.at[pl.ds(0, _CBT * 256)], sem0).wait()
    compute(r30, obft, _CBT)
    pltpu.sync_copy(obft,
                    st_hbm.at[:, pl.ds(_NF * (_CB * 512), _CBT * 512)])


_sc_kernel = functools.partial(
    pl.kernel,
    out_type=(
        jax.ShapeDtypeStruct((_N * 3,), jnp.float32),
        jax.ShapeDtypeStruct((3, _NBLK * 512), jnp.float32),
        jax.ShapeDtypeStruct((_NPAD, 16), jnp.float32),
    ),
    mesh=plsc.VectorSubcoreMesh(
        core_axis_name="c", subcore_axis_name="s",
        num_cores=_NC, num_subcores=_NS),
    compiler_params=pltpu.CompilerParams(
        needs_layout_passes=False, use_tc_tiling_on_sc=False),
    scratch_types=[
        pltpu.VMEM((_RC * 9,), jnp.float32),
        pltpu.VMEM((_RC * 3,), jnp.float32),
        pltpu.VMEM((_RC * 3,), jnp.float32),
        pltpu.VMEM((_RC, 16), jnp.float32),
        pltpu.VMEM((_RC * 3,), jnp.float32),
        pltpu.VMEM((_CB * 256,), jnp.int32),
        pltpu.VMEM((_CB * 256,), jnp.int32),
        pltpu.VMEM((_CB * 256, 16), jnp.float32),
        pltpu.VMEM((_CB * 256, 16), jnp.float32),
        pltpu.VMEM((3, _CB * 512), jnp.float32),
        pltpu.VMEM((3, _CBT * 512), jnp.float32),
        pltpu.SemaphoreType.DMA,
        pltpu.SemaphoreType.DMA,
    ],
)(_sc_body)


def kernel(x, edges, A, b):
  B, N, _ = x.shape
  E = edges.shape[0]
  a9 = A.transpose(0, 2, 3, 1).reshape(9 * N)
  b3 = b.transpose(0, 2, 3, 1).reshape(3 * N)
  x3 = x.transpose(0, 2, 1).reshape(3 * N)
  ev = edges.astype(jnp.int32).reshape(_NBLK, 128, 2)
  ev = ev.transpose(0, 2, 1).reshape(_NBLK * 256)
  nv, st, _unused_tab = _sc_kernel(a9, b3, x3, ev)
  stiffness = (st.reshape(3, _NBLK, 4, 128)
               .transpose(1, 3, 0, 2).reshape(B, E, 3, 4))
  return (nv.reshape(3, N).transpose(1, 0).reshape(B, N, 3), stiffness)


# speedup vs baseline: 1.8749x; 1.0845x over previous
import functools

import jax
import jax.numpy as jnp
from jax import lax
from jax.experimental import pallas as pl
from jax.experimental.pallas import tpu as pltpu
from jax.experimental.pallas import tpu_sc as plsc

_NC = 2
_NS = 16
_NW = _NC * _NS
_L = 16

_N = 50000
_E = 800000
_RT = 3136
_RC = 784
_NPAD = _NS * _RT
_RLAST = _N - (_NS - 1) * _RT - 3 * _RC

_NBLK = _E // 128
_CB = 8
_NQ = -(-_NBLK // _CB)
_CBT = _NBLK - (_NQ - 1) * _CB
_NF = _NQ - 1
_WTAIL = _NF % _NW


def _sc_body(a_hbm, b_hbm, x_hbm, e_hbm, nv_hbm, st_hbm, wtab_hbm,
             av, bv, xv, wrow, nvf, idx0, idx1, r30, r31, obf, obft,
             sem0, sem1):
  cid = lax.axis_index("c")
  sid = lax.axis_index("s")
  wid = sid * _NC + cid
  lane = lax.iota(jnp.int32, _L)

  def build_nv(row_base, nrows):
    pltpu.sync_copy(a_hbm.at[:, pl.ds(row_base, nrows)],
                    av.at[:, pl.ds(0, nrows)])
    pltpu.sync_copy(b_hbm.at[:, pl.ds(row_base, nrows)],
                    bv.at[:, pl.ds(0, nrows)])
    pltpu.sync_copy(x_hbm.at[:, pl.ds(row_base, nrows)],
                    xv.at[:, pl.ds(0, nrows)])

    @plsc.parallel_loop(0, nrows // _L)
    def group(g):
      nid = g * _L + lane
      xs = [xv[j, pl.ds(g * _L, _L)] for j in range(3)]
      for i in range(3):
        bi = bv[i, pl.ds(g * _L, _L)]
        plsc.store_scatter(wrow, [nid, jnp.full((_L,), 9 + i, jnp.int32)], bi)
        acc = bi
        for j in range(3):
          aij = av[3 * i + j, pl.ds(g * _L, _L)]
          plsc.store_scatter(
              wrow, [nid, jnp.full((_L,), 3 * i + j, jnp.int32)], aij)
          acc = acc + aij * xs[j]
        nvf[i, pl.ds(g * _L, _L)] = acc
    pltpu.sync_copy(wrow.at[pl.ds(0, nrows)],
                    wtab_hbm.at[pl.ds(row_base, nrows)])
    pltpu.sync_copy(nvf.at[:, pl.ds(0, nrows)],
                    nv_hbm.at[:, pl.ds(row_base, nrows)])

  with jax.named_scope("p1_build"):
    for r in range(3):
      build_nv(sid * _RT + r * _RC, _RC)

    @pl.when(sid < _NS - 1)
    def _():
      build_nv(sid * _RT + 3 * _RC, _RC)

    @pl.when(sid == _NS - 1)
    def _():
      build_nv(sid * _RT + 3 * _RC, _RLAST)

  with jax.named_scope("p1_barrier"):
    plsc.subcore_barrier()

  def stage(q, idx, sem, rbuf):
    pltpu.sync_copy(e_hbm.at[pl.ds(q * (_CB * 256), _CB * 256)], idx)
    pltpu.async_copy(wtab_hbm.at[idx], rbuf, sem)

  def compute(rbuf, obuf, nblk):
    @plsc.parallel_loop(0, 8)
    def col(ep0):
      for blk in range(nblk):
        rv0 = lane + (blk * 256 + ep0 * 16)
        rv1 = rv0 + 128
        for c in range(12):
          i, j = (c // 3, c % 3) if c < 9 else (c - 9, 3)
          cv = jnp.full((_L,), c, jnp.int32)
          d = (plsc.load_gather(rbuf, [rv0, cv])
               - plsc.load_gather(rbuf, [rv1, cv]))
          obuf[i, pl.ds(blk * 512 + j * 128 + ep0 * 16, _L)] = d * d

  nf = (_NF - wid + _NW - 1) // _NW

  @pl.when(nf > 0)
  def _():
    stage(wid, idx0, sem0, r30)

  def chunk(k, carry):
    q = wid + _NW * k

    def run(idx, sem, rbuf, idxn, semn, rbufn):
      with jax.named_scope("p2_wait"):
        pltpu.make_async_copy(wtab_hbm.at[idx], rbuf, sem).wait()

      with jax.named_scope("p2_stage"):
        @pl.when(k + 1 < nf)
        def _():
          stage(q + _NW, idxn, semn, rbufn)

      with jax.named_scope("p2_compute"):
        compute(rbuf, obf, _CB)
      with jax.named_scope("p2_out"):
        pltpu.sync_copy(obf, st_hbm.at[:, pl.ds(q * (_CB * 512), _CB * 512)])

    @pl.when(k % 2 == 0)
    def _():
      run(idx0, sem0, r30, idx1, sem1, r31)

    @pl.when(k % 2 == 1)
    def _():
      run(idx1, sem1, r31, idx0, sem0, r30)

    return carry

  lax.fori_loop(0, nf, chunk, 0)

  @pl.when(wid == _WTAIL)
  def _():
    pltpu.sync_copy(e_hbm.at[pl.ds(_NF * (_CB * 256), _CBT * 256)],
                    idx0.at[pl.ds(0, _CBT * 256)])
    pltpu.async_copy(wtab_hbm.at[idx0.at[pl.ds(0, _CBT * 256)]],
                     r30.at[pl.ds(0, _CBT * 256)], sem0).wait()
    compute(r30, obft, _CBT)
    pltpu.sync_copy(obft,
                    st_hbm.at[:, pl.ds(_NF * (_CB * 512), _CBT * 512)])


_sc_kernel = functools.partial(
    pl.kernel,
    out_type=(
        jax.ShapeDtypeStruct((3, _N), jnp.float32),
        jax.ShapeDtypeStruct((3, _NBLK * 512), jnp.float32),
        jax.ShapeDtypeStruct((_NPAD, 16), jnp.float32),
    ),
    mesh=plsc.VectorSubcoreMesh(
        core_axis_name="c", subcore_axis_name="s",
        num_cores=_NC, num_subcores=_NS),
    compiler_params=pltpu.CompilerParams(
        needs_layout_passes=False, use_tc_tiling_on_sc=False),
    scratch_types=[
        pltpu.VMEM((9, _RC), jnp.float32),
        pltpu.VMEM((3, _RC), jnp.float32),
        pltpu.VMEM((3, _RC), jnp.float32),
        pltpu.VMEM((_RC, 16), jnp.float32),
        pltpu.VMEM((3, _RC), jnp.float32),
        pltpu.VMEM((_CB * 256,), jnp.int32),
        pltpu.VMEM((_CB * 256,), jnp.int32),
        pltpu.VMEM((_CB * 256, 16), jnp.float32),
        pltpu.VMEM((_CB * 256, 16), jnp.float32),
        pltpu.VMEM((3, _CB * 512), jnp.float32),
        pltpu.VMEM((3, _CBT * 512), jnp.float32),
        pltpu.SemaphoreType.DMA,
        pltpu.SemaphoreType.DMA,
    ],
)(_sc_body)


def kernel(x, edges, A, b):
  B, N, _ = x.shape
  E = edges.shape[0]
  a9 = A.transpose(0, 2, 3, 1).reshape(9, N)
  b3 = b.transpose(0, 2, 3, 1).reshape(3, N)
  x3 = x.transpose(0, 2, 1).reshape(3, N)
  ev = edges.astype(jnp.int32).reshape(_NBLK, 128, 2)
  ev = ev.transpose(0, 2, 1).reshape(_NBLK * 256)
  nv, st, _unused_tab = _sc_kernel(a9, b3, x3, ev)
  stiffness = (st.reshape(3, _NBLK, 4, 128)
               .transpose(1, 3, 0, 2).reshape(B, E, 3, 4))
  return (nv.transpose(1, 0).reshape(B, N, 3), stiffness)


# speedup vs baseline: 2.3202x; 1.2375x over previous
import functools

import jax
import jax.numpy as jnp
from jax import lax
from jax.experimental import pallas as pl
from jax.experimental.pallas import tpu as pltpu
from jax.experimental.pallas import tpu_sc as plsc

_NC = 2
_NS = 16
_NW = _NC * _NS
_L = 16

_N = 50000
_E = 800000
_RT = 3136
_RC = 448
_NSUB = _RT // _RC
_NPAD = _NS * _RT
_RLAST = _N - (_NS - 1) * _RT - (_NSUB - 1) * _RC

_NBLK = _E // 128
_CB = 8
_NQ = -(-_NBLK // _CB)
_CBT = _NBLK - (_NQ - 1) * _CB
_NF = _NQ - 1
_WTAIL = _NF % _NW


def _sc_body(a_hbm, b_hbm, x_hbm, e_hbm, nv_hbm, st_hbm, wtab_hbm,
             av, bv, xv, wrow, nvf, idx0, idx1, r30, r31, obf, obft, dsq,
             sem0, sem1):
  cid = lax.axis_index("c")
  sid = lax.axis_index("s")
  wid = sid * _NC + cid
  lane = lax.iota(jnp.int32, _L)

  def build_nv(row_base, nrows):
    pltpu.sync_copy(a_hbm.at[:, pl.ds(row_base, nrows)],
                    av.at[:, pl.ds(0, nrows)])
    pltpu.sync_copy(b_hbm.at[:, pl.ds(row_base, nrows)],
                    bv.at[:, pl.ds(0, nrows)])
    pltpu.sync_copy(x_hbm.at[:, pl.ds(row_base, nrows)],
                    xv.at[:, pl.ds(0, nrows)])

    @plsc.parallel_loop(0, nrows // _L)
    def group(g):
      nid = g * _L + lane
      xs = [xv[j, pl.ds(g * _L, _L)] for j in range(3)]
      for i in range(3):
        bi = bv[i, pl.ds(g * _L, _L)]
        plsc.store_scatter(wrow, [nid, jnp.full((_L,), 9 + i, jnp.int32)], bi)
        acc = bi
        for j in range(3):
          aij = av[3 * i + j, pl.ds(g * _L, _L)]
          plsc.store_scatter(
              wrow, [nid, jnp.full((_L,), 3 * i + j, jnp.int32)], aij)
          acc = acc + aij * xs[j]
        nvf[i, pl.ds(g * _L, _L)] = acc
    pltpu.sync_copy(wrow.at[pl.ds(0, nrows)],
                    wtab_hbm.at[pl.ds(row_base, nrows)])
    pltpu.sync_copy(nvf.at[:, pl.ds(0, nrows)],
                    nv_hbm.at[:, pl.ds(row_base, nrows)])

  with jax.named_scope("p1_build"):
    for r in range(_NSUB - 1):
      build_nv(sid * _RT + r * _RC, _RC)

    @pl.when(sid < _NS - 1)
    def _():
      build_nv(sid * _RT + (_NSUB - 1) * _RC, _RC)

    @pl.when(sid == _NS - 1)
    def _():
      build_nv(sid * _RT + (_NSUB - 1) * _RC, _RLAST)

  with jax.named_scope("p1_barrier"):
    plsc.subcore_barrier()

  def stage(q, idx, sem, rbuf):
    pltpu.sync_copy(e_hbm.at[pl.ds(q * (_CB * 256), _CB * 256)], idx)
    pltpu.async_copy(wtab_hbm.at[idx], rbuf, sem)

  def compute(rbuf, obuf, nblk):
    @plsc.parallel_loop(0, 128)
    def phase_a(ep):
      for blk in range(nblk):
        r0 = blk * 256 + ep
        d = rbuf[r0] - rbuf[r0 + 128]
        p = blk * 128 + ep
        plsc.store_scatter(dsq, [p * 16 + ((lane + p) & 15)], d * d)

    @plsc.parallel_loop(0, 8)
    def phase_b(ep0):
      lane16 = lane * 16
      for blk in range(nblk):
        p0 = blk * 128 + ep0 * 16
        for c in range(12):
          i, j = (c // 3, c % 3) if c < 9 else (c - 9, 3)
          idx = p0 * 16 + lane16 + ((lane + (c + p0)) & 15)
          v = plsc.load_gather(dsq, [idx])
          obuf[i, pl.ds(blk * 512 + j * 128 + ep0 * 16, _L)] = v

  nf = (_NF - wid + _NW - 1) // _NW

  @pl.when(nf > 0)
  def _():
    stage(wid, idx0, sem0, r30)

  def chunk(k, carry):
    q = wid + _NW * k

    def run(idx, sem, rbuf, idxn, semn, rbufn):
      with jax.named_scope("p2_wait"):
        pltpu.make_async_copy(wtab_hbm.at[idx], rbuf, sem).wait()

      with jax.named_scope("p2_stage"):
        @pl.when(k + 1 < nf)
        def _():
          stage(q + _NW, idxn, semn, rbufn)

      with jax.named_scope("p2_compute"):
        compute(rbuf, obf, _CB)
      with jax.named_scope("p2_out"):
        pltpu.sync_copy(obf, st_hbm.at[:, pl.ds(q * (_CB * 512), _CB * 512)])

    @pl.when(k % 2 == 0)
    def _():
      run(idx0, sem0, r30, idx1, sem1, r31)

    @pl.when(k % 2 == 1)
    def _():
      run(idx1, sem1, r31, idx0, sem0, r30)

    return carry

  lax.fori_loop(0, nf, chunk, 0)

  @pl.when(wid == _WTAIL)
  def _():
    pltpu.sync_copy(e_hbm.at[pl.ds(_NF * (_CB * 256), _CBT * 256)],
                    idx0.at[pl.ds(0, _CBT * 256)])
    pltpu.async_copy(wtab_hbm.at[idx0.at[pl.ds(0, _CBT * 256)]],
                     r30.at[pl.ds(0, _CBT * 256)], sem0).wait()
    compute(r30, obft, _CBT)
    pltpu.sync_copy(obft,
                    st_hbm.at[:, pl.ds(_NF * (_CB * 512), _CBT * 512)])


_sc_kernel = functools.partial(
    pl.kernel,
    out_type=(
        jax.ShapeDtypeStruct((3, _N), jnp.float32),
        jax.ShapeDtypeStruct((3, _NBLK * 512), jnp.float32),
        jax.ShapeDtypeStruct((_NPAD, 16), jnp.float32),
    ),
    mesh=plsc.VectorSubcoreMesh(
        core_axis_name="c", subcore_axis_name="s",
        num_cores=_NC, num_subcores=_NS),
    compiler_params=pltpu.CompilerParams(
        needs_layout_passes=False, use_tc_tiling_on_sc=False),
    scratch_types=[
        pltpu.VMEM((9, _RC), jnp.float32),
        pltpu.VMEM((3, _RC), jnp.float32),
        pltpu.VMEM((3, _RC), jnp.float32),
        pltpu.VMEM((_RC, 16), jnp.float32),
        pltpu.VMEM((3, _RC), jnp.float32),
        pltpu.VMEM((_CB * 256,), jnp.int32),
        pltpu.VMEM((_CB * 256,), jnp.int32),
        pltpu.VMEM((_CB * 256, 16), jnp.float32),
        pltpu.VMEM((_CB * 256, 16), jnp.float32),
        pltpu.VMEM((3, _CB * 512), jnp.float32),
        pltpu.VMEM((3, _CBT * 512), jnp.float32),
        pltpu.VMEM((_CB * 128 * 16,), jnp.float32),
        pltpu.SemaphoreType.DMA,
        pltpu.SemaphoreType.DMA,
    ],
)(_sc_body)


def kernel(x, edges, A, b):
  B, N, _ = x.shape
  E = edges.shape[0]
  a9 = A.transpose(0, 2, 3, 1).reshape(9, N)
  b3 = b.transpose(0, 2, 3, 1).reshape(3, N)
  x3 = x.transpose(0, 2, 1).reshape(3, N)
  ev = edges.astype(jnp.int32).reshape(_NBLK, 128, 2)
  ev = ev.transpose(0, 2, 1).reshape(_NBLK * 256)
  nv, st, _unused_tab = _sc_kernel(a9, b3, x3, ev)
  stiffness = (st.reshape(3, _NBLK, 4, 128)
               .transpose(1, 3, 0, 2).reshape(B, E, 3, 4))
  return (nv.transpose(1, 0).reshape(B, N, 3), stiffness)


# speedup vs baseline: 2.4342x; 1.0492x over previous
import functools

import jax
import jax.numpy as jnp
from jax import lax
from jax.experimental import pallas as pl
from jax.experimental.pallas import tpu as pltpu
from jax.experimental.pallas import tpu_sc as plsc

_NC = 2
_NS = 16
_NW = _NC * _NS
_L = 16

_N = 50000
_E = 800000
_RT = 3136
_RC = 448
_NSUB = _RT // _RC
_NPAD = _NS * _RT
_RLAST = _N - (_NS - 1) * _RT - (_NSUB - 1) * _RC

_NBLK = _E // 128
_CB = 8
_NQ = -(-_NBLK // _CB)
_CBT = _NBLK - (_NQ - 1) * _CB
_NF = _NQ - 1
_WTAIL = _NF % _NW


def _sc_body(a_hbm, b_hbm, x_hbm, e_hbm, nv_hbm, st_hbm, wtab_hbm,
             av, bv, xv, wrow, nvf, idx0, idx1, r30, r31, obf, obft, dsq,
             sem0, sem1, sem2):
  cid = lax.axis_index("c")
  sid = lax.axis_index("s")
  wid = sid * _NC + cid
  lane = lax.iota(jnp.int32, _L)

  def build_nv(row_base, nrows):
    cpa = pltpu.async_copy(a_hbm.at[:, pl.ds(row_base, nrows)],
                           av.at[:, pl.ds(0, nrows)], sem1)
    cpb = pltpu.async_copy(b_hbm.at[:, pl.ds(row_base, nrows)],
                           bv.at[:, pl.ds(0, nrows)], sem1)
    cpx = pltpu.async_copy(x_hbm.at[:, pl.ds(row_base, nrows)],
                           xv.at[:, pl.ds(0, nrows)], sem1)
    cpa.wait()
    cpb.wait()
    cpx.wait()

    @plsc.parallel_loop(0, nrows // _L)
    def group(g):
      nid = g * _L + lane
      xs = [xv[j, pl.ds(g * _L, _L)] for j in range(3)]
      for i in range(3):
        bi = bv[i, pl.ds(g * _L, _L)]
        plsc.store_scatter(wrow, [nid, jnp.full((_L,), 9 + i, jnp.int32)], bi)
        acc = bi
        for j in range(3):
          aij = av[3 * i + j, pl.ds(g * _L, _L)]
          plsc.store_scatter(
              wrow, [nid, jnp.full((_L,), 3 * i + j, jnp.int32)], aij)
          acc = acc + aij * xs[j]
        nvf[i, pl.ds(g * _L, _L)] = acc
    pltpu.sync_copy(wrow.at[pl.ds(0, nrows)],
                    wtab_hbm.at[pl.ds(row_base, nrows)])
    pltpu.sync_copy(nvf.at[:, pl.ds(0, nrows)],
                    nv_hbm.at[:, pl.ds(row_base, nrows)])

  with jax.named_scope("p1_build"):
    for r in range(_NSUB - 1):
      build_nv(sid * _RT + r * _RC, _RC)

    @pl.when(sid < _NS - 1)
    def _():
      build_nv(sid * _RT + (_NSUB - 1) * _RC, _RC)

    @pl.when(sid == _NS - 1)
    def _():
      build_nv(sid * _RT + (_NSUB - 1) * _RC, _RLAST)

  with jax.named_scope("p1_barrier"):
    plsc.subcore_barrier()

  def stage(q, idx, sem, rbuf):
    pltpu.sync_copy(e_hbm.at[pl.ds(q * (_CB * 256), _CB * 256)], idx)
    pltpu.async_copy(wtab_hbm.at[idx], rbuf, sem)

  def compute(rbuf, obuf, nblk):
    @plsc.parallel_loop(0, 128)
    def phase_a(ep):
      for blk in range(nblk):
        r0 = blk * 256 + ep
        d = rbuf[r0] - rbuf[r0 + 128]
        p = blk * 128 + ep
        plsc.store_scatter(dsq, [p * 16 + ((lane + p) & 15)], d * d)

    @plsc.parallel_loop(0, 8)
    def phase_b(ep0):
      lane16 = lane * 16
      for blk in range(nblk):
        p0 = blk * 128 + ep0 * 16
        for c in range(12):
          i, j = (c // 3, c % 3) if c < 9 else (c - 9, 3)
          idx = p0 * 16 + lane16 + ((lane + (c + p0)) & 15)
          v = plsc.load_gather(dsq, [idx])
          obuf[i, pl.ds(blk * 512 + j * 128 + ep0 * 16, _L)] = v

  nf = (_NF - wid + _NW - 1) // _NW

  @pl.when(nf > 0)
  def _():
    stage(wid, idx0, sem0, r30)

  def chunk(k, carry):
    q = wid + _NW * k

    def run(idx, sem, rbuf, idxn, semn, rbufn):
      with jax.named_scope("p2_wait"):
        pltpu.make_async_copy(wtab_hbm.at[idx], rbuf, sem).wait()

      with jax.named_scope("p2_stage"):
        @pl.when(k + 1 < nf)
        def _():
          stage(q + _NW, idxn, semn, rbufn)

      with jax.named_scope("p2_owait"):
        @pl.when(k > 0)
        def _():
          pltpu.make_async_copy(
              obf, st_hbm.at[:, pl.ds((q - _NW) * (_CB * 512), _CB * 512)],
              sem2).wait()

      with jax.named_scope("p2_compute"):
        compute(rbuf, obf, _CB)
      with jax.named_scope("p2_out"):
        pltpu.async_copy(
            obf, st_hbm.at[:, pl.ds(q * (_CB * 512), _CB * 512)], sem2)

    @pl.when(k % 2 == 0)
    def _():
      run(idx0, sem0, r30, idx1, sem1, r31)

    @pl.when(k % 2 == 1)
    def _():
      run(idx1, sem1, r31, idx0, sem0, r30)

    return carry

  lax.fori_loop(0, nf, chunk, 0)

  @pl.when(nf > 0)
  def _():
    qlast = wid + _NW * (nf - 1)
    pltpu.make_async_copy(
        obf, st_hbm.at[:, pl.ds(qlast * (_CB * 512), _CB * 512)],
        sem2).wait()

  @pl.when(wid == _WTAIL)
  def _():
    pltpu.sync_copy(e_hbm.at[pl.ds(_NF * (_CB * 256), _CBT * 256)],
                    idx0.at[pl.ds(0, _CBT * 256)])
    pltpu.async_copy(wtab_hbm.at[idx0.at[pl.ds(0, _CBT * 256)]],
                     r30.at[pl.ds(0, _CBT * 256)], sem0).wait()
    compute(r30, obft, _CBT)
    pltpu.sync_copy(obft,
                    st_hbm.at[:, pl.ds(_NF * (_CB * 512), _CBT * 512)])


_sc_kernel = functools.partial(
    pl.kernel,
    out_type=(
        jax.ShapeDtypeStruct((3, _N), jnp.float32),
        jax.ShapeDtypeStruct((3, _NBLK * 512), jnp.float32),
        jax.ShapeDtypeStruct((_NPAD, 16), jnp.float32),
    ),
    mesh=plsc.VectorSubcoreMesh(
        core_axis_name="c", subcore_axis_name="s",
        num_cores=_NC, num_subcores=_NS),
    compiler_params=pltpu.CompilerParams(
        needs_layout_passes=False, use_tc_tiling_on_sc=False),
    scratch_types=[
        pltpu.VMEM((9, _RC), jnp.float32),
        pltpu.VMEM((3, _RC), jnp.float32),
        pltpu.VMEM((3, _RC), jnp.float32),
        pltpu.VMEM((_RC, 16), jnp.float32),
        pltpu.VMEM((3, _RC), jnp.float32),
        pltpu.VMEM((_CB * 256,), jnp.int32),
        pltpu.VMEM((_CB * 256,), jnp.int32),
        pltpu.VMEM((_CB * 256, 16), jnp.float32),
        pltpu.VMEM((_CB * 256, 16), jnp.float32),
        pltpu.VMEM((3, _CB * 512), jnp.float32),
        pltpu.VMEM((3, _CBT * 512), jnp.float32),
        pltpu.VMEM((_CB * 128 * 16,), jnp.float32),
        pltpu.SemaphoreType.DMA,
        pltpu.SemaphoreType.DMA,
        pltpu.SemaphoreType.DMA,
    ],
)(_sc_body)


def kernel(x, edges, A, b):
  B, N, _ = x.shape
  E = edges.shape[0]
  a9 = A.transpose(0, 2, 3, 1).reshape(9, N)
  b3 = b.transpose(0, 2, 3, 1).reshape(3, N)
  x3 = x.transpose(0, 2, 1).reshape(3, N)
  ev = edges.astype(jnp.int32).reshape(_NBLK, 128, 2)
  ev = ev.transpose(0, 2, 1).reshape(_NBLK * 256)
  nv, st, _unused_tab = _sc_kernel(a9, b3, x3, ev)
  stiffness = (st.reshape(3, _NBLK, 4, 128)
               .transpose(1, 3, 0, 2).reshape(B, E, 3, 4))
  return (nv.transpose(1, 0).reshape(B, N, 3), stiffness)


# speedup vs baseline: 2.4446x; 1.0042x over previous
import functools

import jax
import jax.numpy as jnp
from jax import lax
from jax.experimental import pallas as pl
from jax.experimental.pallas import tpu as pltpu
from jax.experimental.pallas import tpu_sc as plsc

_NC = 2
_NS = 16
_NW = _NC * _NS
_L = 16

_N = 50000
_E = 800000
_RT = 3136
_RC = 448
_NSUB = _RT // _RC
_NPAD = _NS * _RT
_RLAST = _N - (_NS - 1) * _RT - (_NSUB - 1) * _RC

_NBLK = _E // 128
_CB = 8
_NQ = -(-_NBLK // _CB)
_CBT = _NBLK - (_NQ - 1) * _CB
_NF = _NQ - 1
_WTAIL = _NF % _NW


def _sc_body(a_hbm, b_hbm, x_hbm, e_hbm, nv_hbm, st_hbm, wtab_hbm,
             av, bv, xv, wrow, nvf, idx0, idx1, r30, r31, obf, obft, dsq,
             sem0, sem1, sem2):
  cid = lax.axis_index("c")
  sid = lax.axis_index("s")
  wid = sid * _NC + cid
  lane = lax.iota(jnp.int32, _L)

  def build_nv(row_base, nrows):
    cpa = pltpu.async_copy(a_hbm.at[:, pl.ds(row_base, nrows)],
                           av.at[:, pl.ds(0, nrows)], sem1)
    cpb = pltpu.async_copy(b_hbm.at[:, pl.ds(row_base, nrows)],
                           bv.at[:, pl.ds(0, nrows)], sem1)
    cpx = pltpu.async_copy(x_hbm.at[:, pl.ds(row_base, nrows)],
                           xv.at[:, pl.ds(0, nrows)], sem1)
    cpa.wait()
    cpb.wait()
    cpx.wait()

    @plsc.parallel_loop(0, nrows // _L)
    def group(g):
      nid = g * _L + lane
      xs = [xv[j, pl.ds(g * _L, _L)] for j in range(3)]
      for i in range(3):
        bi = bv[i, pl.ds(g * _L, _L)]
        plsc.store_scatter(wrow, [nid, jnp.full((_L,), 9 + i, jnp.int32)], bi)
        acc = bi
        for j in range(3):
          aij = av[3 * i + j, pl.ds(g * _L, _L)]
          plsc.store_scatter(
              wrow, [nid, jnp.full((_L,), 3 * i + j, jnp.int32)], aij)
          acc = acc + aij * xs[j]
        nvf[i, pl.ds(g * _L, _L)] = acc
    pltpu.sync_copy(wrow.at[pl.ds(0, nrows)],
                    wtab_hbm.at[pl.ds(row_base, nrows)])
    pltpu.sync_copy(nvf.at[:, pl.ds(0, nrows)],
                    nv_hbm.at[:, pl.ds(row_base, nrows)])

  with jax.named_scope("p1_build"):
    for r in range(_NSUB - 1):
      build_nv(sid * _RT + r * _RC, _RC)

    @pl.when(sid < _NS - 1)
    def _():
      build_nv(sid * _RT + (_NSUB - 1) * _RC, _RC)

    @pl.when(sid == _NS - 1)
    def _():
      build_nv(sid * _RT + (_NSUB - 1) * _RC, _RLAST)

  with jax.named_scope("p1_barrier"):
    plsc.subcore_barrier()

  def stage(q, idx, sem, rbuf):
    pltpu.sync_copy(e_hbm.at[pl.ds(q * (_CB * 256), _CB * 256)], idx)
    pltpu.async_copy(wtab_hbm.at[idx], rbuf, sem)

  def compute(rbuf, obuf, nblk):
    @plsc.parallel_loop(0, 128, unroll=4)
    def phase_a(ep):
      for blk in range(nblk):
        r0 = blk * 256 + ep
        d = rbuf[r0] - rbuf[r0 + 128]
        p = blk * 128 + ep
        plsc.store_scatter(dsq, [p * 16 + ((lane + p) & 15)], d * d)

    @plsc.parallel_loop(0, 8)
    def phase_b(ep0):
      lane16 = lane * 16
      for blk in range(nblk):
        p0 = blk * 128 + ep0 * 16
        for c in range(12):
          i, j = (c // 3, c % 3) if c < 9 else (c - 9, 3)
          idx = p0 * 16 + lane16 + ((lane + (c + p0)) & 15)
          v = plsc.load_gather(dsq, [idx])
          obuf[i, pl.ds(blk * 512 + j * 128 + ep0 * 16, _L)] = v

  nf = (_NF - wid + _NW - 1) // _NW

  @pl.when(nf > 0)
  def _():
    stage(wid, idx0, sem0, r30)

  def chunk(k, carry):
    q = wid + _NW * k

    def run(idx, sem, rbuf, idxn, semn, rbufn):
      with jax.named_scope("p2_wait"):
        pltpu.make_async_copy(wtab_hbm.at[idx], rbuf, sem).wait()

      with jax.named_scope("p2_stage"):
        @pl.when(k + 1 < nf)
        def _():
          stage(q + _NW, idxn, semn, rbufn)

      with jax.named_scope("p2_owait"):
        @pl.when(k > 0)
        def _():
          pltpu.make_async_copy(
              obf, st_hbm.at[:, pl.ds((q - _NW) * (_CB * 512), _CB * 512)],
              sem2).wait()

      with jax.named_scope("p2_compute"):
        compute(rbuf, obf, _CB)
      with jax.named_scope("p2_out"):
        pltpu.async_copy(
            obf, st_hbm.at[:, pl.ds(q * (_CB * 512), _CB * 512)], sem2)

    @pl.when(k % 2 == 0)
    def _():
      run(idx0, sem0, r30, idx1, sem1, r31)

    @pl.when(k % 2 == 1)
    def _():
      run(idx1, sem1, r31, idx0, sem0, r30)

    return carry

  lax.fori_loop(0, nf, chunk, 0)

  @pl.when(nf > 0)
  def _():
    qlast = wid + _NW * (nf - 1)
    pltpu.make_async_copy(
        obf, st_hbm.at[:, pl.ds(qlast * (_CB * 512), _CB * 512)],
        sem2).wait()

  @pl.when(wid == _WTAIL)
  def _():
    pltpu.sync_copy(e_hbm.at[pl.ds(_NF * (_CB * 256), _CBT * 256)],
                    idx0.at[pl.ds(0, _CBT * 256)])
    pltpu.async_copy(wtab_hbm.at[idx0.at[pl.ds(0, _CBT * 256)]],
                     r30.at[pl.ds(0, _CBT * 256)], sem0).wait()
    compute(r30, obft, _CBT)
    pltpu.sync_copy(obft,
                    st_hbm.at[:, pl.ds(_NF * (_CB * 512), _CBT * 512)])


_sc_kernel = functools.partial(
    pl.kernel,
    out_type=(
        jax.ShapeDtypeStruct((3, _N), jnp.float32),
        jax.ShapeDtypeStruct((3, _NBLK * 512), jnp.float32),
        jax.ShapeDtypeStruct((_NPAD, 16), jnp.float32),
    ),
    mesh=plsc.VectorSubcoreMesh(
        core_axis_name="c", subcore_axis_name="s",
        num_cores=_NC, num_subcores=_NS),
    compiler_params=pltpu.CompilerParams(
        needs_layout_passes=False, use_tc_tiling_on_sc=False),
    scratch_types=[
        pltpu.VMEM((9, _RC), jnp.float32),
        pltpu.VMEM((3, _RC), jnp.float32),
        pltpu.VMEM((3, _RC), jnp.float32),
        pltpu.VMEM((_RC, 16), jnp.float32),
        pltpu.VMEM((3, _RC), jnp.float32),
        pltpu.VMEM((_CB * 256,), jnp.int32),
        pltpu.VMEM((_CB * 256,), jnp.int32),
        pltpu.VMEM((_CB * 256, 16), jnp.float32),
        pltpu.VMEM((_CB * 256, 16), jnp.float32),
        pltpu.VMEM((3, _CB * 512), jnp.float32),
        pltpu.VMEM((3, _CBT * 512), jnp.float32),
        pltpu.VMEM((_CB * 128 * 16,), jnp.float32),
        pltpu.SemaphoreType.DMA,
        pltpu.SemaphoreType.DMA,
        pltpu.SemaphoreType.DMA,
    ],
)(_sc_body)


def kernel(x, edges, A, b):
  B, N, _ = x.shape
  E = edges.shape[0]
  a9 = A.transpose(0, 2, 3, 1).reshape(9, N)
  b3 = b.transpose(0, 2, 3, 1).reshape(3, N)
  x3 = x.transpose(0, 2, 1).reshape(3, N)
  ev = edges.astype(jnp.int32).reshape(_NBLK, 128, 2)
  ev = ev.transpose(0, 2, 1).reshape(_NBLK * 256)
  nv, st, _unused_tab = _sc_kernel(a9, b3, x3, ev)
  stiffness = (st.reshape(3, _NBLK, 4, 128)
               .transpose(1, 3, 0, 2).reshape(B, E, 3, 4))
  return (nv.transpose(1, 0).reshape(B, N, 3), stiffness)
